# baseline (speedup 1.0000x reference)
import functools

import jax
import jax.numpy as jnp
from jax import lax
from jax.experimental import pallas as pl
from jax.experimental.pallas import tpu as pltpu
from jax.experimental.pallas import tpu_sc as plsc

D = 128
NE = 10000
E = 320000
HALF = E // 2
NQ = 80000
NACC = 10240
DUMMY = 10016
NC = 2
NS = 16
E_PT = HALF // NS
Q_PT = NQ // NS
CH = 128
E_CHN = 80
Q_CHN = 40
STG = 40
NB = 2
ZR = NACC // NS
OR_ = NACC // NS

_f32 = jnp.float32
_i32 = jnp.int32


@functools.lru_cache(maxsize=None)
def _mesh():
    return plsc.VectorSubcoreMesh(core_axis_name="c", subcore_axis_name="s",
                                  num_cores=NC, num_subcores=NS)


def _zero_acc(acc, zbuf, zsem, s):
    zero = jnp.zeros((16,), _f32)
    for r in range(32):
        for col in range(D // 16):
            zbuf[r, pl.ds(col * 16, 16)] = zero
    nz = ZR // 32
    for k in range(nz):
        pltpu.async_copy(zbuf, acc.at[pl.ds(s * ZR + k * 32, 32)], zsem)
    for k in range(nz):
        pltpu.make_async_copy(zbuf, acc.at[pl.ds(s * ZR, 32)], zsem).wait()


def _chunk_loop(table, idx4, acc, gi, si, rows, sem, tid, n_ch):
    def step(j, carry):
        pltpu.sync_copy(idx4.at[tid, j, 0], gi)
        pltpu.sync_copy(idx4.at[tid, j, 1], si)
        pltpu.async_copy(table.at[gi], rows, sem).wait()
        pltpu.sync_copy(rows, acc.at[si], add=True)
        return carry
    lax.fori_loop(0, n_ch, step, 0)


def _copy_out(acc, out, c, s):
    pltpu.sync_copy(acc.at[pl.ds(s * OR_, OR_)], out.at[c, pl.ds(s * OR_, OR_)])


@functools.lru_cache(maxsize=None)
def _sc_pass(do_edge: bool, do_qual: bool):
    scratch = [
        pltpu.VMEM_SHARED((NACC, D), _f32),
        pltpu.VMEM((32, D), _f32),
        pltpu.VMEM((CH,), _i32),
        pltpu.VMEM((CH,), _i32),
        pltpu.VMEM((CH, D), _f32),
        pltpu.SemaphoreType.DMA,
        pltpu.SemaphoreType.DMA,
    ]

    def body(*refs):
        if do_edge and do_qual:
            table_e, eidx4, table_q, qidx4, out = refs[:5]
            rest = refs[5:]
        elif do_edge:
            table_e, eidx4, out = refs[:3]
            rest = refs[3:]
        else:
            table_q, qidx4, out = refs[:3]
            rest = refs[3:]
        acc, zbuf, gi, si, rows, sem, zsem = rest
        c = lax.axis_index("c")
        s = lax.axis_index("s")
        tid = c * NS + s
        _zero_acc(acc, zbuf, zsem, s)
        plsc.subcore_barrier()
        if do_edge:
            _chunk_loop(table_e, eidx4, acc, gi, si, rows, sem, tid, E_CHN)
        if do_qual:
            _chunk_loop(table_q, qidx4, acc, gi, si, rows, sem, tid, Q_CHN)
        plsc.subcore_barrier()
        _copy_out(acc, out, c, s)

    return pl.kernel(
        body,
        out_type=jax.ShapeDtypeStruct((NC, NACC, D), _f32),
        mesh=_mesh(),
        scratch_types=scratch,
    )


def _sc_edge_pass(table_e, eidx4):
    return _sc_pass(True, False)(table_e, eidx4)


def _sc_edge_qual_pass(table_e, eidx4, table_q, qidx4):
    return _sc_pass(True, True)(table_e, eidx4, table_q, qidx4)


def _sc_qual_pass(table_q, qidx4):
    return _sc_pass(False, True)(table_q, qidx4)


BR = 1024
NBLK = NACC // BR


def _dense_body(acc2, ssq, x, w7, bias8, wq, out, xq):
    t = jnp.dot(acc2[0], w7[0], preferred_element_type=_f32)
    t += jnp.dot(acc2[1], w7[1], preferred_element_type=_f32)
    t += jnp.dot(ssq[0], w7[2], preferred_element_type=_f32)
    t += jnp.dot(ssq[1], w7[3], preferred_element_type=_f32)
    t += jnp.dot(ssq[2], w7[4], preferred_element_type=_f32)
    t += jnp.dot(ssq[3], w7[5], preferred_element_type=_f32)
    t += jnp.dot(x[...], w7[6], preferred_element_type=_f32)
    t += bias8[0:1, :]
    o = jnp.tanh(t)
    out[...] = o
    xq[...] = jnp.dot(o, wq[...], preferred_element_type=_f32)


_dense_call = pl.pallas_call(
    _dense_body,
    grid=(NBLK,),
    in_specs=[
        pl.BlockSpec((NC, BR, D), lambda i: (0, i, 0)),
        pl.BlockSpec((4, BR, D), lambda i: (0, i, 0)),
        pl.BlockSpec((BR, D), lambda i: (i, 0)),
        pl.BlockSpec((7, D, D), lambda i: (0, 0, 0)),
        pl.BlockSpec((8, D), lambda i: (0, 0)),
        pl.BlockSpec((D, D), lambda i: (0, 0)),
    ],
    out_specs=[
        pl.BlockSpec((BR, D), lambda i: (i, 0)),
        pl.BlockSpec((BR, D), lambda i: (i, 0)),
    ],
    out_shape=[
        jax.ShapeDtypeStruct((NACC, D), _f32),
        jax.ShapeDtypeStruct((NACC, D), _f32),
    ],
)


def _relout_body(rel, m, out):
    out[...] = jnp.dot(rel[...], m[...], preferred_element_type=_f32)


_relout_call = pl.pallas_call(
    _relout_body,
    out_shape=jax.ShapeDtypeStruct((400, D), _f32),
)


GB = 4096
GQ = GB * 6


@functools.lru_cache(maxsize=None)
def _sc_gather_kernel():
    def body(x1, x3, r1, ent_ix, rel_ix, q_ents, q_rels,
             o_ent, o_rel, o_qobj, o_qrel,
             gi, rows, sem):
        c = lax.axis_index("c")
        s = lax.axis_index("s")
        wid = c * NS + s
        jobs = [
            (x1, ent_ix, o_ent, GB // (NC * NS * 128)),
            (r1, rel_ix, o_rel, GB // (NC * NS * 128)),
            (x3, q_ents, o_qobj, GQ // (NC * NS * 128)),
            (r1, q_rels, o_qrel, GQ // (NC * NS * 128)),
        ]
        for table, idx_hbm, out_hbm, nch in jobs:
            for j in range(nch):
                off = wid * (nch * 128) + j * 128
                pltpu.sync_copy(idx_hbm.at[pl.ds(off, 128)], gi)
                pltpu.async_copy(table.at[gi], rows, sem).wait()
                pltpu.sync_copy(rows, out_hbm.at[pl.ds(off, 128)])

    return pl.kernel(
        body,
        out_type=[
            jax.ShapeDtypeStruct((GB, D), _f32),
            jax.ShapeDtypeStruct((GB, D), _f32),
            jax.ShapeDtypeStruct((GQ, D), _f32),
            jax.ShapeDtypeStruct((GQ, D), _f32),
        ],
        mesh=_mesh(),
        scratch_types=[
            pltpu.VMEM((128,), _i32),
            pltpu.VMEM((128, D), _f32),
            pltpu.SemaphoreType.DMA,
        ],
    )


def _sc_gather_out(*args):
    return _sc_gather_kernel()(*args)


def _pad_rows(a, n):
    return jnp.concatenate([a, jnp.zeros((n - a.shape[0],) + a.shape[1:], a.dtype)], 0)


def kernel(ent_ix, rel_ix, quals_ix, ent_embs, rel_embs, edge_index, edge_type, quals, params):
    src = edge_index[0].astype(_i32)
    dst = edge_index[1].astype(_i32)
    etype = edge_type.astype(_i32)
    qr = quals[0].astype(_i32)
    qe = quals[1].astype(_i32)
    qedge = quals[2].astype(_i32)

    qdst_raw = jnp.take(dst, qedge)
    in_m = qedge < HALF

    def tile3(a, tiles, per, chunks, fill):
        a2 = a.reshape(tiles, per)
        a2 = jnp.pad(a2, ((0, 0), (0, chunks * CH - per)), constant_values=fill)
        return a2.reshape(tiles, chunks, CH).astype(_i32)

    gsrc3 = tile3(src, NC * NS, E_PT, E_CHN, 0)
    getype3 = tile3(etype, NC * NS, E_PT, E_CHN, 0)
    sdst3 = tile3(dst, NC * NS, E_PT, E_CHN, DUMMY)
    qe3 = jnp.tile(tile3(qe, NS, Q_PT, Q_CHN, 0), (NC, 1, 1))
    qr3 = jnp.tile(tile3(qr, NS, Q_PT, Q_CHN, 0), (NC, 1, 1))
    qdst3 = tile3(jnp.concatenate([
        jnp.where(in_m, qdst_raw, DUMMY),
        jnp.where(in_m, DUMMY, qdst_raw),
    ]), NC * NS, Q_PT, Q_CHN, DUMMY)
    eidx4 = jnp.stack([gsrc3, sdst3], axis=2)
    etidx4 = jnp.stack([getype3, sdst3], axis=2)
    qidx4 = jnp.stack([qe3, qdst3], axis=2)
    qridx4 = jnp.stack([qr3, qdst3], axis=2)

    deg_in = jnp.zeros((NE,), _f32).at[dst[:HALF]].add(1.0)
    deg_out = jnp.zeros((NE,), _f32).at[dst[HALF:]].add(1.0)
    s_in = _pad_rows((1.0 / (3.0 * jnp.maximum(deg_in, 1.0)))[:, None], NACC)[:, 0]
    s_out = _pad_rows((1.0 / (3.0 * jnp.maximum(deg_out, 1.0)))[:, None], NACC)[:, 0]
    sbc = jnp.stack([s_in, s_out])[:, :, None]

    S2 = _sc_edge_pass(rel_embs, etidx4)
    Sq2 = _sc_qual_pass(rel_embs, qridx4)
    ssq = jnp.stack([
        S2[0] * s_in[:, None], S2[1] * s_out[:, None],
        Sq2[0] * s_in[:, None], Sq2[1] * s_out[:, None],
    ])

    p = params
    eye = jnp.eye(D, dtype=_f32)
    Mr1 = p["trip1"]["w_rel"]
    Mr2 = Mr1 @ p["qual1"]["w_rel"]
    Mr3 = Mr2 @ p["both1"]["w_rel"]
    Mr1b = Mr2 @ p["trip2"]["w_rel"]
    Mr2b = Mr3 @ p["qual2"]["w_rel"]
    zero_w = jnp.zeros((D, D), _f32)

    def layer_weights(pk, M, with_q, wq_next):
        w_in, w_out, w_loop = pk["w_in"], pk["w_out"], pk["w_loop"]
        if with_q:
            wq = pk["w_q"]
            aq_in, aq_out = -(M @ wq @ w_in), -(M @ wq @ w_out)
        else:
            aq_in, aq_out = zero_w, zero_w
        w7 = jnp.stack([w_in, w_out, -(M @ w_in), -(M @ w_out), aq_in, aq_out,
                        w_loop / 3.0])
        bias8 = jnp.tile(pk["bias"][None, :] - (pk["loop_rel"] @ w_loop) / 3.0, (8, 1))
        return w7, bias8, (wq_next if wq_next is not None else zero_w)

    x0 = _pad_rows(ent_embs, NACC)

    def run_layer(x, xq, pk, M, with_q, wq_next):
        if with_q:
            acc2 = _sc_edge_qual_pass(x, eidx4, xq, qidx4)
        else:
            acc2 = _sc_edge_pass(x, eidx4)
        w7, bias8, wqn = layer_weights(pk, M, with_q, wq_next)
        return _dense_call(acc2 * sbc, ssq, x, w7, bias8, wqn)

    x1, xq1 = run_layer(x0, None, p["trip1"], eye, False, p["qual1"]["w_q"])
    x2, xq2 = run_layer(x1, xq1, p["qual1"], Mr1, True, p["both1"]["w_q"])
    x3, xq3 = run_layer(x2, xq2, p["both1"], Mr2, True, p["qual2"]["w_q"])
    x1b, _ = run_layer(x2, None, p["trip2"], Mr2, False, None)
    x2b, xq5 = run_layer(x3, xq3, p["qual2"], Mr3, True, p["both2"]["w_q"])
    x3b, _ = run_layer(x2b, xq5, p["both2"], Mr2b, True, None)

    r1 = _relout_call(rel_embs, Mr1b)

    quals_ents = quals_ix[:, 1::2].reshape(-1).astype(_i32)
    quals_rels = quals_ix[:, 0::2].reshape(-1).astype(_i32)
    o_ent, o_rel, o_qobj, o_qrel = _sc_gather_out(
        x1b, x3b, r1, ent_ix.astype(_i32), rel_ix.astype(_i32),
        quals_ents, quals_rels)

    return (o_ent, o_rel, o_qobj.reshape(GB, 6, D), o_qrel.reshape(GB, 6, D),
            x1b[:NE], r1)


# speedup vs baseline: 1.3761x; 1.3761x over previous
import functools

import jax
import jax.numpy as jnp
from jax import lax
from jax.experimental import pallas as pl
from jax.experimental.pallas import tpu as pltpu
from jax.experimental.pallas import tpu_sc as plsc

D = 128
NE = 10000
E = 320000
HALF = E // 2
NQ = 80000
NACC = 10240
DUMMY = 10016
NC = 2
NS = 16
E_PT = HALF // NS
Q_PT = NQ // NS
ECH = 80
QCH = 40
ZR = NACC // NS
OR_ = NACC // NS

_f32 = jnp.float32
_i32 = jnp.int32


@functools.lru_cache(maxsize=None)
def _mesh():
    return plsc.VectorSubcoreMesh(core_axis_name="c", subcore_axis_name="s",
                                  num_cores=NC, num_subcores=NS)


def _zero_acc(acc, zbuf, zsem, s):
    zero = jnp.zeros((16,), _f32)
    for r in range(32):
        for col in range(D // 16):
            zbuf[r, pl.ds(col * 16, 16)] = zero
    nz = ZR // 32
    for k in range(nz):
        pltpu.async_copy(zbuf, acc.at[pl.ds(s * ZR + k * 32, 32)], zsem)
    for k in range(nz):
        pltpu.make_async_copy(zbuf, acc.at[pl.ds(s * ZR, 32)], zsem).wait()


def _chunk_loop(table, gidx, sidx, acc, gi, si, rows, sem, gbase, sbase, ch, n_ch):
    def step(j, carry):
        pltpu.sync_copy(gidx.at[pl.ds(gbase + j * ch, ch)], gi)
        pltpu.async_copy(table.at[gi], rows, sem).wait()
        pltpu.sync_copy(sidx.at[pl.ds(sbase + j * ch, ch)], si)
        pltpu.sync_copy(rows, acc.at[si], add=True)
        return carry
    lax.fori_loop(0, n_ch, step, 0)


def _copy_out(acc, out, c, s):
    pltpu.sync_copy(acc.at[pl.ds(s * OR_, OR_)], out.at[c, pl.ds(s * OR_, OR_)])


@functools.lru_cache(maxsize=None)
def _sc_pass(do_edge: bool, do_qual: bool):
    scratch = [
        pltpu.VMEM_SHARED((NACC, D), _f32),
        pltpu.VMEM((32, D), _f32),
        pltpu.VMEM((ECH,), _i32),
        pltpu.VMEM((ECH,), _i32),
        pltpu.VMEM((ECH, D), _f32),
        pltpu.VMEM((QCH,), _i32),
        pltpu.VMEM((QCH,), _i32),
        pltpu.VMEM((QCH, D), _f32),
        pltpu.SemaphoreType.DMA,
        pltpu.SemaphoreType.DMA,
    ]

    def body(*refs):
        if do_edge and do_qual:
            table_e, gidx_e, sidx_e, table_q, gidx_q, qdst2, out = refs[:7]
            rest = refs[7:]
        elif do_edge:
            table_e, gidx_e, sidx_e, out = refs[:4]
            rest = refs[4:]
        else:
            table_q, gidx_q, qdst2, out = refs[:4]
            rest = refs[4:]
        acc, zbuf, gi_e, si_e, rows_e, gi_q, si_q, rows_q, sem, zsem = rest
        c = lax.axis_index("c")
        s = lax.axis_index("s")
        _zero_acc(acc, zbuf, zsem, s)
        plsc.subcore_barrier()
        if do_edge:
            ebase = c * HALF + s * E_PT
            _chunk_loop(table_e, gidx_e, sidx_e, acc, gi_e, si_e, rows_e, sem,
                        ebase, ebase, ECH, E_PT // ECH)
        if do_qual:
            _chunk_loop(table_q, gidx_q, qdst2, acc, gi_q, si_q, rows_q, sem,
                        s * Q_PT, c * NQ + s * Q_PT, QCH, Q_PT // QCH)
        plsc.subcore_barrier()
        _copy_out(acc, out, c, s)

    return pl.kernel(
        body,
        out_type=jax.ShapeDtypeStruct((NC, NACC, D), _f32),
        mesh=_mesh(),
        scratch_types=scratch,
    )


def _sc_edge_pass(table_e, gidx_e, sidx_e):
    return _sc_pass(True, False)(table_e, gidx_e, sidx_e)


def _sc_edge_qual_pass(table_e, gidx_e, sidx_e, table_q, gidx_q, qdst2):
    return _sc_pass(True, True)(table_e, gidx_e, sidx_e, table_q, gidx_q, qdst2)


def _sc_qual_pass(table_q, gidx_q, qdst2):
    return _sc_pass(False, True)(table_q, gidx_q, qdst2)


BR = 1024
NBLK = NACC // BR


def _dense_body(acc2, ssq, x, w7, bias8, wq, out, xq):
    t = jnp.dot(acc2[0], w7[0], preferred_element_type=_f32)
    t += jnp.dot(acc2[1], w7[1], preferred_element_type=_f32)
    t += jnp.dot(ssq[0], w7[2], preferred_element_type=_f32)
    t += jnp.dot(ssq[1], w7[3], preferred_element_type=_f32)
    t += jnp.dot(ssq[2], w7[4], preferred_element_type=_f32)
    t += jnp.dot(ssq[3], w7[5], preferred_element_type=_f32)
    t += jnp.dot(x[...], w7[6], preferred_element_type=_f32)
    t += bias8[0:1, :]
    o = jnp.tanh(t)
    out[...] = o
    xq[...] = jnp.dot(o, wq[...], preferred_element_type=_f32)


_dense_call = pl.pallas_call(
    _dense_body,
    grid=(NBLK,),
    in_specs=[
        pl.BlockSpec((NC, BR, D), lambda i: (0, i, 0)),
        pl.BlockSpec((4, BR, D), lambda i: (0, i, 0)),
        pl.BlockSpec((BR, D), lambda i: (i, 0)),
        pl.BlockSpec((7, D, D), lambda i: (0, 0, 0)),
        pl.BlockSpec((8, D), lambda i: (0, 0)),
        pl.BlockSpec((D, D), lambda i: (0, 0)),
    ],
    out_specs=[
        pl.BlockSpec((BR, D), lambda i: (i, 0)),
        pl.BlockSpec((BR, D), lambda i: (i, 0)),
    ],
    out_shape=[
        jax.ShapeDtypeStruct((NACC, D), _f32),
        jax.ShapeDtypeStruct((NACC, D), _f32),
    ],
)


def _relout_body(rel, m, out):
    out[...] = jnp.dot(rel[...], m[...], preferred_element_type=_f32)


_relout_call = pl.pallas_call(
    _relout_body,
    out_shape=jax.ShapeDtypeStruct((400, D), _f32),
)


GB = 4096
GQ = GB * 6


@functools.lru_cache(maxsize=None)
def _sc_gather_kernel():
    def body(x1, x3, r1, ent_ix, rel_ix, q_ents, q_rels,
             o_ent, o_rel, o_qobj, o_qrel,
             gi, rows, sem):
        c = lax.axis_index("c")
        s = lax.axis_index("s")
        wid = c * NS + s
        jobs = [
            (x1, ent_ix, o_ent, GB // (NC * NS * 128)),
            (r1, rel_ix, o_rel, GB // (NC * NS * 128)),
            (x3, q_ents, o_qobj, GQ // (NC * NS * 128)),
            (r1, q_rels, o_qrel, GQ // (NC * NS * 128)),
        ]
        for table, idx_hbm, out_hbm, nch in jobs:
            for j in range(nch):
                off = wid * (nch * 128) + j * 128
                pltpu.sync_copy(idx_hbm.at[pl.ds(off, 128)], gi)
                pltpu.async_copy(table.at[gi], rows, sem).wait()
                pltpu.sync_copy(rows, out_hbm.at[pl.ds(off, 128)])

    return pl.kernel(
        body,
        out_type=[
            jax.ShapeDtypeStruct((GB, D), _f32),
            jax.ShapeDtypeStruct((GB, D), _f32),
            jax.ShapeDtypeStruct((GQ, D), _f32),
            jax.ShapeDtypeStruct((GQ, D), _f32),
        ],
        mesh=_mesh(),
        scratch_types=[
            pltpu.VMEM((128,), _i32),
            pltpu.VMEM((128, D), _f32),
            pltpu.SemaphoreType.DMA,
        ],
    )


def _sc_gather_out(*args):
    return _sc_gather_kernel()(*args)


def _pad_rows(a, n):
    return jnp.concatenate([a, jnp.zeros((n - a.shape[0],) + a.shape[1:], a.dtype)], 0)


def kernel(ent_ix, rel_ix, quals_ix, ent_embs, rel_embs, edge_index, edge_type, quals, params):
    src = edge_index[0].astype(_i32)
    dst = edge_index[1].astype(_i32)
    etype = edge_type.astype(_i32)
    qr = quals[0].astype(_i32)
    qe = quals[1].astype(_i32)
    qedge = quals[2].astype(_i32)

    qdst_raw = jnp.take(dst, qedge)
    in_m = qedge < HALF
    qdst2 = jnp.concatenate([
        jnp.where(in_m, qdst_raw, DUMMY),
        jnp.where(in_m, DUMMY, qdst_raw),
    ]).astype(_i32)

    deg_in = jnp.zeros((NE,), _f32).at[dst[:HALF]].add(1.0)
    deg_out = jnp.zeros((NE,), _f32).at[dst[HALF:]].add(1.0)
    s_in = _pad_rows((1.0 / (3.0 * jnp.maximum(deg_in, 1.0)))[:, None], NACC)[:, 0]
    s_out = _pad_rows((1.0 / (3.0 * jnp.maximum(deg_out, 1.0)))[:, None], NACC)[:, 0]
    sbc = jnp.stack([s_in, s_out])[:, :, None]

    S2 = _sc_edge_pass(rel_embs, etype, dst)
    Sq2 = _sc_qual_pass(rel_embs, qr, qdst2)
    ssq = jnp.stack([
        S2[0] * s_in[:, None], S2[1] * s_out[:, None],
        Sq2[0] * s_in[:, None], Sq2[1] * s_out[:, None],
    ])

    p = params
    eye = jnp.eye(D, dtype=_f32)
    Mr1 = p["trip1"]["w_rel"]
    Mr2 = Mr1 @ p["qual1"]["w_rel"]
    Mr3 = Mr2 @ p["both1"]["w_rel"]
    Mr1b = Mr2 @ p["trip2"]["w_rel"]
    Mr2b = Mr3 @ p["qual2"]["w_rel"]
    zero_w = jnp.zeros((D, D), _f32)

    def layer_weights(pk, M, with_q, wq_next):
        w_in, w_out, w_loop = pk["w_in"], pk["w_out"], pk["w_loop"]
        if with_q:
            wq = pk["w_q"]
            aq_in, aq_out = -(M @ wq @ w_in), -(M @ wq @ w_out)
        else:
            aq_in, aq_out = zero_w, zero_w
        w7 = jnp.stack([w_in, w_out, -(M @ w_in), -(M @ w_out), aq_in, aq_out,
                        w_loop / 3.0])
        bias8 = jnp.tile(pk["bias"][None, :] - (pk["loop_rel"] @ w_loop) / 3.0, (8, 1))
        return w7, bias8, (wq_next if wq_next is not None else zero_w)

    x0 = _pad_rows(ent_embs, NACC)

    def run_layer(x, xq, pk, M, with_q, wq_next):
        if with_q:
            acc2 = _sc_edge_qual_pass(x, src, dst, xq, qe, qdst2)
        else:
            acc2 = _sc_edge_pass(x, src, dst)
        w7, bias8, wqn = layer_weights(pk, M, with_q, wq_next)
        return _dense_call(acc2 * sbc, ssq, x, w7, bias8, wqn)

    x1, xq1 = run_layer(x0, None, p["trip1"], eye, False, p["qual1"]["w_q"])
    x2, xq2 = run_layer(x1, xq1, p["qual1"], Mr1, True, p["both1"]["w_q"])
    x3, xq3 = run_layer(x2, xq2, p["both1"], Mr2, True, p["qual2"]["w_q"])
    x1b, _ = run_layer(x2, None, p["trip2"], Mr2, False, None)
    x2b, xq5 = run_layer(x3, xq3, p["qual2"], Mr3, True, p["both2"]["w_q"])
    x3b, _ = run_layer(x2b, xq5, p["both2"], Mr2b, True, None)

    r1 = _relout_call(rel_embs, Mr1b)

    quals_ents = quals_ix[:, 1::2].reshape(-1).astype(_i32)
    quals_rels = quals_ix[:, 0::2].reshape(-1).astype(_i32)
    o_ent, o_rel, o_qobj, o_qrel = _sc_gather_out(
        x1b, x3b, r1, ent_ix.astype(_i32), rel_ix.astype(_i32),
        quals_ents, quals_rels)

    return (o_ent, o_rel, o_qobj.reshape(GB, 6, D), o_qrel.reshape(GB, 6, D),
            x1b[:NE], r1)


# speedup vs baseline: 1.9037x; 1.3834x over previous
import functools

import jax
import jax.numpy as jnp
from jax import lax
from jax.experimental import pallas as pl
from jax.experimental.pallas import tpu as pltpu
from jax.experimental.pallas import tpu_sc as plsc

D = 128
NE = 10000
E = 320000
HALF = E // 2
NQ = 80000
NACC = 10240
DUMMY = 10016
NC = 2
NS = 16
E_PT = HALF // NS
Q_PT = NQ // NS
ECH = 80
QCH = 40
ZR = NACC // NS
OR_ = NACC // NS

_f32 = jnp.float32
_i32 = jnp.int32


@functools.lru_cache(maxsize=None)
def _mesh():
    return plsc.VectorSubcoreMesh(core_axis_name="c", subcore_axis_name="s",
                                  num_cores=NC, num_subcores=NS)


def _zero_acc(acc, zbuf, zsem, s):
    zero = jnp.zeros((16,), _f32)
    for r in range(32):
        for col in range(D // 16):
            zbuf[r, pl.ds(col * 16, 16)] = zero
    nz = ZR // 32
    for k in range(nz):
        pltpu.async_copy(zbuf, acc.at[pl.ds(s * ZR + k * 32, 32)], zsem)
    for k in range(nz):
        pltpu.make_async_copy(zbuf, acc.at[pl.ds(s * ZR, 32)], zsem).wait()


def _chunk_loop(table, gidx, sidx, acc, gi2, si2, rows2, sems, gbase, sbase, ch, n_ch):
    assert n_ch % 2 == 1

    def gstart(j, b):
        pltpu.sync_copy(gidx.at[pl.ds(gbase + j * ch, ch)], gi2[b])
        pltpu.async_copy(table.at[gi2[b]], rows2[b], sems[b])

    def gwait(b):
        pltpu.make_async_copy(table.at[gi2[b]], rows2[b], sems[b]).wait()

    def put(j, b):
        pltpu.sync_copy(sidx.at[pl.ds(sbase + j * ch, ch)], si2[b])
        pltpu.sync_copy(rows2[b], acc.at[si2[b]], add=True)

    gstart(0, 0)

    def pair(g, carry):
        for b in (0, 1):
            k = g * 2 + b
            gwait(b)
            gstart(k + 1, 1 - b)
            put(k, b)
        return carry

    lax.fori_loop(0, n_ch // 2, pair, 0)
    gwait(0)
    put(n_ch - 1, 0)


def _copy_out(acc, out, c, s):
    pltpu.sync_copy(acc.at[pl.ds(s * OR_, OR_)], out.at[c, pl.ds(s * OR_, OR_)])


@functools.lru_cache(maxsize=None)
def _sc_pass(do_edge: bool, do_qual: bool):
    scratch = [
        pltpu.VMEM_SHARED((NACC, D), _f32),
        pltpu.VMEM((32, D), _f32),
        pltpu.VMEM((ECH,), _i32),
        pltpu.VMEM((ECH,), _i32),
        pltpu.VMEM((ECH,), _i32),
        pltpu.VMEM((ECH,), _i32),
        pltpu.VMEM((ECH, D), _f32),
        pltpu.VMEM((ECH, D), _f32),
        pltpu.VMEM((QCH,), _i32),
        pltpu.VMEM((QCH,), _i32),
        pltpu.VMEM((QCH,), _i32),
        pltpu.VMEM((QCH,), _i32),
        pltpu.VMEM((QCH, D), _f32),
        pltpu.VMEM((QCH, D), _f32),
        pltpu.SemaphoreType.DMA,
        pltpu.SemaphoreType.DMA,
        pltpu.SemaphoreType.DMA,
    ]

    def body(*refs):
        if do_edge and do_qual:
            table_e, gidx_e, sidx_e, table_q, gidx_q, qdst2, out = refs[:7]
            rest = refs[7:]
        elif do_edge:
            table_e, gidx_e, sidx_e, out = refs[:4]
            rest = refs[4:]
        else:
            table_q, gidx_q, qdst2, out = refs[:4]
            rest = refs[4:]
        (acc, zbuf, gie0, gie1, sie0, sie1, rowse0, rowse1,
         giq0, giq1, siq0, siq1, rowsq0, rowsq1, sem0, sem1, zsem) = rest
        c = lax.axis_index("c")
        s = lax.axis_index("s")
        _zero_acc(acc, zbuf, zsem, s)
        plsc.subcore_barrier()
        if do_edge:
            ebase = c * HALF + s * E_PT
            _chunk_loop(table_e, gidx_e, sidx_e, acc, [gie0, gie1],
                        [sie0, sie1], [rowse0, rowse1], [sem0, sem1],
                        ebase, ebase, ECH, E_PT // ECH)
        if do_qual:
            _chunk_loop(table_q, gidx_q, qdst2, acc, [giq0, giq1],
                        [siq0, siq1], [rowsq0, rowsq1], [sem0, sem1],
                        s * Q_PT, c * NQ + s * Q_PT, QCH, Q_PT // QCH)
        plsc.subcore_barrier()
        _copy_out(acc, out, c, s)

    return pl.kernel(
        body,
        out_type=jax.ShapeDtypeStruct((NC, NACC, D), _f32),
        mesh=_mesh(),
        scratch_types=scratch,
    )


def _sc_edge_pass(table_e, gidx_e, sidx_e):
    return _sc_pass(True, False)(table_e, gidx_e, sidx_e)


def _sc_edge_qual_pass(table_e, gidx_e, sidx_e, table_q, gidx_q, qdst2):
    return _sc_pass(True, True)(table_e, gidx_e, sidx_e, table_q, gidx_q, qdst2)


def _sc_qual_pass(table_q, gidx_q, qdst2):
    return _sc_pass(False, True)(table_q, gidx_q, qdst2)


BR = 1024
NBLK = NACC // BR


def _dense_body(acc2, ssq, x, w7, bias8, wq, out, xq):
    t = jnp.dot(acc2[0], w7[0], preferred_element_type=_f32)
    t += jnp.dot(acc2[1], w7[1], preferred_element_type=_f32)
    t += jnp.dot(ssq[0], w7[2], preferred_element_type=_f32)
    t += jnp.dot(ssq[1], w7[3], preferred_element_type=_f32)
    t += jnp.dot(ssq[2], w7[4], preferred_element_type=_f32)
    t += jnp.dot(ssq[3], w7[5], preferred_element_type=_f32)
    t += jnp.dot(x[...], w7[6], preferred_element_type=_f32)
    t += bias8[0:1, :]
    o = jnp.tanh(t)
    out[...] = o
    xq[...] = jnp.dot(o, wq[...], preferred_element_type=_f32)


_dense_call = pl.pallas_call(
    _dense_body,
    grid=(NBLK,),
    in_specs=[
        pl.BlockSpec((NC, BR, D), lambda i: (0, i, 0)),
        pl.BlockSpec((4, BR, D), lambda i: (0, i, 0)),
        pl.BlockSpec((BR, D), lambda i: (i, 0)),
        pl.BlockSpec((7, D, D), lambda i: (0, 0, 0)),
        pl.BlockSpec((8, D), lambda i: (0, 0)),
        pl.BlockSpec((D, D), lambda i: (0, 0)),
    ],
    out_specs=[
        pl.BlockSpec((BR, D), lambda i: (i, 0)),
        pl.BlockSpec((BR, D), lambda i: (i, 0)),
    ],
    out_shape=[
        jax.ShapeDtypeStruct((NACC, D), _f32),
        jax.ShapeDtypeStruct((NACC, D), _f32),
    ],
)


def _relout_body(rel, m, out):
    out[...] = jnp.dot(rel[...], m[...], preferred_element_type=_f32)


_relout_call = pl.pallas_call(
    _relout_body,
    out_shape=jax.ShapeDtypeStruct((400, D), _f32),
)


GB = 4096
GQ = GB * 6


@functools.lru_cache(maxsize=None)
def _sc_gather_kernel():
    def body(x1, x3, r1, ent_ix, rel_ix, q_ents, q_rels,
             o_ent, o_rel, o_qobj, o_qrel,
             gi, rows, sem):
        c = lax.axis_index("c")
        s = lax.axis_index("s")
        wid = c * NS + s
        jobs = [
            (x1, ent_ix, o_ent, GB // (NC * NS * 128)),
            (r1, rel_ix, o_rel, GB // (NC * NS * 128)),
            (x3, q_ents, o_qobj, GQ // (NC * NS * 128)),
            (r1, q_rels, o_qrel, GQ // (NC * NS * 128)),
        ]
        for table, idx_hbm, out_hbm, nch in jobs:
            for j in range(nch):
                off = wid * (nch * 128) + j * 128
                pltpu.sync_copy(idx_hbm.at[pl.ds(off, 128)], gi)
                pltpu.async_copy(table.at[gi], rows, sem).wait()
                pltpu.sync_copy(rows, out_hbm.at[pl.ds(off, 128)])

    return pl.kernel(
        body,
        out_type=[
            jax.ShapeDtypeStruct((GB, D), _f32),
            jax.ShapeDtypeStruct((GB, D), _f32),
            jax.ShapeDtypeStruct((GQ, D), _f32),
            jax.ShapeDtypeStruct((GQ, D), _f32),
        ],
        mesh=_mesh(),
        scratch_types=[
            pltpu.VMEM((128,), _i32),
            pltpu.VMEM((128, D), _f32),
            pltpu.SemaphoreType.DMA,
        ],
    )


def _sc_gather_out(*args):
    return _sc_gather_kernel()(*args)


def _pad_rows(a, n):
    return jnp.concatenate([a, jnp.zeros((n - a.shape[0],) + a.shape[1:], a.dtype)], 0)


def kernel(ent_ix, rel_ix, quals_ix, ent_embs, rel_embs, edge_index, edge_type, quals, params):
    src = edge_index[0].astype(_i32)
    dst = edge_index[1].astype(_i32)
    etype = edge_type.astype(_i32)
    qr = quals[0].astype(_i32)
    qe = quals[1].astype(_i32)
    qedge = quals[2].astype(_i32)

    qdst_raw = jnp.take(dst, qedge)
    in_m = qedge < HALF
    qdst2 = jnp.concatenate([
        jnp.where(in_m, qdst_raw, DUMMY),
        jnp.where(in_m, DUMMY, qdst_raw),
    ]).astype(_i32)

    deg_in = jnp.zeros((NE,), _f32).at[dst[:HALF]].add(1.0)
    deg_out = jnp.zeros((NE,), _f32).at[dst[HALF:]].add(1.0)
    s_in = _pad_rows((1.0 / (3.0 * jnp.maximum(deg_in, 1.0)))[:, None], NACC)[:, 0]
    s_out = _pad_rows((1.0 / (3.0 * jnp.maximum(deg_out, 1.0)))[:, None], NACC)[:, 0]
    sbc = jnp.stack([s_in, s_out])[:, :, None]

    S2 = _sc_edge_pass(rel_embs, etype, dst)
    Sq2 = _sc_qual_pass(rel_embs, qr, qdst2)
    ssq = jnp.stack([
        S2[0] * s_in[:, None], S2[1] * s_out[:, None],
        Sq2[0] * s_in[:, None], Sq2[1] * s_out[:, None],
    ])

    p = params
    eye = jnp.eye(D, dtype=_f32)
    Mr1 = p["trip1"]["w_rel"]
    Mr2 = Mr1 @ p["qual1"]["w_rel"]
    Mr3 = Mr2 @ p["both1"]["w_rel"]
    Mr1b = Mr2 @ p["trip2"]["w_rel"]
    Mr2b = Mr3 @ p["qual2"]["w_rel"]
    zero_w = jnp.zeros((D, D), _f32)

    def layer_weights(pk, M, with_q, wq_next):
        w_in, w_out, w_loop = pk["w_in"], pk["w_out"], pk["w_loop"]
        if with_q:
            wq = pk["w_q"]
            aq_in, aq_out = -(M @ wq @ w_in), -(M @ wq @ w_out)
        else:
            aq_in, aq_out = zero_w, zero_w
        w7 = jnp.stack([w_in, w_out, -(M @ w_in), -(M @ w_out), aq_in, aq_out,
                        w_loop / 3.0])
        bias8 = jnp.tile(pk["bias"][None, :] - (pk["loop_rel"] @ w_loop) / 3.0, (8, 1))
        return w7, bias8, (wq_next if wq_next is not None else zero_w)

    x0 = _pad_rows(ent_embs, NACC)

    def run_layer(x, xq, pk, M, with_q, wq_next):
        if with_q:
            acc2 = _sc_edge_qual_pass(x, src, dst, xq, qe, qdst2)
        else:
            acc2 = _sc_edge_pass(x, src, dst)
        w7, bias8, wqn = layer_weights(pk, M, with_q, wq_next)
        return _dense_call(acc2 * sbc, ssq, x, w7, bias8, wqn)

    x1, xq1 = run_layer(x0, None, p["trip1"], eye, False, p["qual1"]["w_q"])
    x2, xq2 = run_layer(x1, xq1, p["qual1"], Mr1, True, p["both1"]["w_q"])
    x3, xq3 = run_layer(x2, xq2, p["both1"], Mr2, True, p["qual2"]["w_q"])
    x1b, _ = run_layer(x2, None, p["trip2"], Mr2, False, None)
    x2b, xq5 = run_layer(x3, xq3, p["qual2"], Mr3, True, p["both2"]["w_q"])
    x3b, _ = run_layer(x2b, xq5, p["both2"], Mr2b, True, None)

    r1 = _relout_call(rel_embs, Mr1b)

    quals_ents = quals_ix[:, 1::2].reshape(-1).astype(_i32)
    quals_rels = quals_ix[:, 0::2].reshape(-1).astype(_i32)
    o_ent, o_rel, o_qobj, o_qrel = _sc_gather_out(
        x1b, x3b, r1, ent_ix.astype(_i32), rel_ix.astype(_i32),
        quals_ents, quals_rels)

    return (o_ent, o_rel, o_qobj.reshape(GB, 6, D), o_qrel.reshape(GB, 6, D),
            x1b[:NE], r1)
